# baseline (speedup 1.0000x reference)
import jax
import jax.numpy as jnp
from jax import lax
from jax.experimental import pallas as pl
from jax.experimental.pallas import tpu as pltpu
from jax.experimental.pallas import tpu_sc as plsc

_B = 64
_T = 2048
_E = 128
_NW = 32
_TW = _T // _NW
_LANES = 16
_CR = 2 * _TW
_NC = _B // 2


def _emb_body(idx_hbm, tok_hbm, pos_hbm, out_hbm,
              idx_v, pos_v, rows0, rows1, rows2, rows3,
              sg0, sg1, sg2, sg3, ss0, ss1, ss2, ss3):
    c = lax.axis_index("c")
    s = lax.axis_index("s")
    wid = s * 2 + c
    t0 = wid * _TW

    bufs = (rows0, rows1, rows2, rows3)
    sgs = (sg0, sg1, sg2, sg3)
    sss = (ss0, ss1, ss2, ss3)

    pltpu.sync_copy(pos_hbm.at[pl.ds(t0, _TW)], pos_v)
    pltpu.sync_copy(idx_hbm.at[wid], idx_v)

    def gather_start(m, p):
        pltpu.async_copy(
            tok_hbm.at[idx_v.at[pl.ds(m * _CR, _CR)]], bufs[p], sgs[p])

    def add_pos(rows):
        for half in range(2):
            @plsc.parallel_loop(0, _TW, step=1, unroll=4)
            def _(i):
                for j in range(_E // _LANES):
                    sl = pl.ds(j * _LANES, _LANES)
                    plsc.addupdate(rows.at[half * _TW + i, sl], pos_v[i, sl])

    def process(m, p):
        pltpu.make_async_copy(
            tok_hbm.at[idx_v.at[pl.ds(m * _CR, _CR)]], bufs[p], sgs[p]).wait()
        add_pos(bufs[p])
        pltpu.async_copy(bufs[p].at[pl.ds(0, _TW)],
                         out_hbm.at[pl.ds((2 * m) * _T + t0, _TW)], sss[p])
        pltpu.async_copy(bufs[p].at[pl.ds(_TW, _TW)],
                         out_hbm.at[pl.ds((2 * m + 1) * _T + t0, _TW)], sss[p])

    def store_wait(m, p):
        pltpu.make_async_copy(
            out_hbm.at[pl.ds(m * _T + t0, _CR)], bufs[p], sss[p]).wait()

    gather_start(0, 0)
    gather_start(1, 1)
    gather_start(2, 2)
    process(0, 0)
    gather_start(3, 3)
    process(1, 1)

    def quad(j, carry):
        for o in range(4):
            m = 4 * j + o
            store_wait(m - 4, o)
            gather_start(m, o)
            process(m - 2, (o + 2) % 4)
        return carry

    lax.fori_loop(1, _NC // 4, quad, 0)

    process(_NC - 2, (_NC - 2) % 4)
    process(_NC - 1, (_NC - 1) % 4)
    for o in range(4):
        store_wait(_NC - 4 + o, o)


@jax.jit
def _emb(idx_r, tok_table, pos_table):
    mesh = plsc.VectorSubcoreMesh(core_axis_name="c", subcore_axis_name="s")
    f = pl.kernel(
        _emb_body,
        out_type=jax.ShapeDtypeStruct((_B * _T, _E), jnp.float32),
        mesh=mesh,
        scratch_types=(
            [pltpu.VMEM((_B * _TW,), jnp.int32)]
            + [pltpu.VMEM((_TW, _E), jnp.float32)]
            + [pltpu.VMEM((_CR, _E), jnp.float32)] * 4
            + [pltpu.SemaphoreType.DMA] * 8
        ),
    )
    return f(idx_r, tok_table, pos_table)


def kernel(idx, tok_table, pos_table):
    idx_r = (idx.astype(jnp.int32)
             .reshape(_B, _NW, _TW).transpose(1, 0, 2).reshape(_NW, _B * _TW))
    out = _emb(idx_r, tok_table, pos_table)
    return out.reshape(_B, _T, _E)


# speedup vs baseline: 1.0083x; 1.0083x over previous
import jax
import jax.numpy as jnp
from jax import lax
from jax.experimental import pallas as pl
from jax.experimental.pallas import tpu as pltpu
from jax.experimental.pallas import tpu_sc as plsc

_B = 64
_T = 2048
_E = 128
_NW = 32
_TW = _T // _NW
_LANES = 16
_CR = 2 * _TW
_NC = _B // 2


def _emb_body(idx_hbm, tok_hbm, pos_hbm, out_hbm,
              idx_v, pos_v, rows0, rows1, rows2, rows3,
              si, sg0, sg1, sg2, sg3, ss0, ss1, ss2, ss3):
    c = lax.axis_index("c")
    s = lax.axis_index("s")
    wid = s * 2 + c
    t0 = wid * _TW

    bufs = (rows0, rows1, rows2, rows3)
    sgs = (sg0, sg1, sg2, sg3)
    sss = (ss0, ss1, ss2, ss3)

    for b in range(_B):
        pltpu.async_copy(idx_hbm.at[pl.ds(b * _T + t0, _TW)],
                         idx_v.at[pl.ds(b * _TW, _TW)], si)
    pltpu.sync_copy(pos_hbm.at[pl.ds(t0, _TW)], pos_v)
    pltpu.make_async_copy(idx_hbm.at[pl.ds(0, _B * _TW)], idx_v, si).wait()

    def gather_start(m, p):
        pltpu.async_copy(
            tok_hbm.at[idx_v.at[pl.ds(m * _CR, _CR)]], bufs[p], sgs[p])

    def add_pos(rows):
        for half in range(2):
            @plsc.parallel_loop(0, _TW, step=1, unroll=4)
            def _(i):
                for j in range(_E // _LANES):
                    sl = pl.ds(j * _LANES, _LANES)
                    plsc.addupdate(rows.at[half * _TW + i, sl], pos_v[i, sl])

    def process(m, p):
        pltpu.make_async_copy(
            tok_hbm.at[idx_v.at[pl.ds(m * _CR, _CR)]], bufs[p], sgs[p]).wait()
        add_pos(bufs[p])
        pltpu.async_copy(bufs[p].at[pl.ds(0, _TW)],
                         out_hbm.at[pl.ds((2 * m) * _T + t0, _TW)], sss[p])
        pltpu.async_copy(bufs[p].at[pl.ds(_TW, _TW)],
                         out_hbm.at[pl.ds((2 * m + 1) * _T + t0, _TW)], sss[p])

    def store_wait(m, p):
        pltpu.make_async_copy(
            out_hbm.at[pl.ds(m * _T + t0, _CR)], bufs[p], sss[p]).wait()

    gather_start(0, 0)
    gather_start(1, 1)
    gather_start(2, 2)
    process(0, 0)
    gather_start(3, 3)
    process(1, 1)

    def quad(j, carry):
        for o in range(4):
            m = 4 * j + o
            store_wait(m - 4, o)
            gather_start(m, o)
            process(m - 2, (o + 2) % 4)
        return carry

    lax.fori_loop(1, _NC // 4, quad, 0)

    process(_NC - 2, (_NC - 2) % 4)
    process(_NC - 1, (_NC - 1) % 4)
    for o in range(4):
        store_wait(_NC - 4 + o, o)


@jax.jit
def _emb(idx_r, tok_table, pos_table):
    mesh = plsc.VectorSubcoreMesh(core_axis_name="c", subcore_axis_name="s")
    f = pl.kernel(
        _emb_body,
        out_type=jax.ShapeDtypeStruct((_B * _T, _E), jnp.float32),
        mesh=mesh,
        scratch_types=(
            [pltpu.VMEM((_B * _TW,), jnp.int32)]
            + [pltpu.VMEM((_TW, _E), jnp.float32)]
            + [pltpu.VMEM((_CR, _E), jnp.float32)] * 4
            + [pltpu.SemaphoreType.DMA] * 9
        ),
    )
    return f(idx_r, tok_table, pos_table)


def kernel(idx, tok_table, pos_table):
    idx_flat = idx.astype(jnp.int32).reshape(_B * _T)
    out = _emb(idx_flat, tok_table, pos_table)
    return out.reshape(_B, _T, _E)


# speedup vs baseline: 1.0215x; 1.0130x over previous
import jax
import jax.numpy as jnp
from jax import lax
from jax.experimental import pallas as pl
from jax.experimental.pallas import tpu as pltpu
from jax.experimental.pallas import tpu_sc as plsc

_B = 64
_T = 2048
_E = 128
_NW = 32
_TW = _T // _NW
_LANES = 16
_WIN = 2 * _TW


def _emb_body(idx_hbm, tok_hbm, pos_hbm, out_hbm,
              idx_v, pos_v, rows0, rows1, rows2, rows3,
              si, sg0, sg1, sg2, sg3, ss0, ss1, ss2, ss3):
    c = lax.axis_index("c")
    s = lax.axis_index("s")
    t0 = s * _WIN + c * _TW

    bufs = (rows0, rows1, rows2, rows3)
    sgs = (sg0, sg1, sg2, sg3)
    sss = (ss0, ss1, ss2, ss3)

    for b in range(_B):
        pltpu.async_copy(idx_hbm.at[b, pl.ds(s * _WIN, _WIN)],
                         idx_v.at[b], si)
    pltpu.sync_copy(pos_hbm.at[pl.ds(t0, _TW)], pos_v)
    pltpu.make_async_copy(idx_hbm.at[:, pl.ds(0, _WIN)], idx_v, si).wait()

    def gather_start(k, p):
        pltpu.async_copy(
            tok_hbm.at[idx_v.at[k, pl.ds(c * _TW, _TW)]], bufs[p], sgs[p])

    def add_pos(rows):
        @plsc.parallel_loop(0, _TW, step=1, unroll=2)
        def _(i):
            for j in range(_E // _LANES):
                sl = pl.ds(j * _LANES, _LANES)
                plsc.addupdate(rows.at[i, sl], pos_v[i, sl])

    def process(k, p):
        pltpu.make_async_copy(
            tok_hbm.at[idx_v.at[k, pl.ds(c * _TW, _TW)]], bufs[p],
            sgs[p]).wait()
        add_pos(bufs[p])
        pltpu.async_copy(bufs[p], out_hbm.at[pl.ds(k * _T + t0, _TW)], sss[p])

    def store_wait(k, p):
        pltpu.make_async_copy(
            bufs[p], out_hbm.at[pl.ds(k * _T + t0, _TW)], sss[p]).wait()

    gather_start(0, 0)
    gather_start(1, 1)
    gather_start(2, 2)
    process(0, 0)
    gather_start(3, 3)
    process(1, 1)

    def quad(j, carry):
        for o in range(4):
            k = 4 * j + o
            store_wait(k - 4, o)
            gather_start(k, o)
            process(k - 2, (o + 2) % 4)
        return carry

    lax.fori_loop(1, _B // 4, quad, 0)

    process(_B - 2, (_B - 2) % 4)
    process(_B - 1, (_B - 1) % 4)
    for o in range(4):
        store_wait(_B - 4 + o, o)


@jax.jit
def _emb(idx, tok_table, pos_table):
    mesh = plsc.VectorSubcoreMesh(core_axis_name="c", subcore_axis_name="s")
    f = pl.kernel(
        _emb_body,
        out_type=jax.ShapeDtypeStruct((_B * _T, _E), jnp.float32),
        mesh=mesh,
        scratch_types=(
            [pltpu.VMEM((_B, _WIN), jnp.int32)]
            + [pltpu.VMEM((_TW, _E), jnp.float32)] * 5
            + [pltpu.SemaphoreType.DMA] * 9
        ),
    )
    return f(idx, tok_table, pos_table)


def kernel(idx, tok_table, pos_table):
    out = _emb(idx.astype(jnp.int32), tok_table, pos_table)
    return out.reshape(_B, _T, _E)
